# baseline (speedup 1.0000x reference)
import functools

import jax
import jax.numpy as jnp
from jax import lax
from jax.experimental import pallas as pl
from jax.experimental.pallas import tpu as pltpu
from jax.experimental.pallas import tpu_sc as plsc

B = 4096
CTX = 20
NOISE = 26
VD = 128
NW = 32
NB = B // NW
SUB = 8
NSUB = NB // SUB
IROW = 32
CROWS = NB * CTX // IROW

_NG = B * NOISE
GC = 128
NOGB = 4

_MESH = plsc.VectorSubcoreMesh(core_axis_name="c", subcore_axis_name="s")


@functools.partial(
    pl.kernel,
    mesh=_MESH,
    out_type=[
        jax.ShapeDtypeStruct((B, VD), jnp.float32),
        jax.ShapeDtypeStruct((_NG, VD), jnp.float32),
    ],
    scratch_types=[
        pltpu.VMEM((1, 128), jnp.int32),
        pltpu.VMEM((CROWS, IROW), jnp.int32),
        pltpu.VMEM((NOISE, GC), jnp.int32),
        pltpu.VMEM((SUB, VD), jnp.float32),
        pltpu.VMEM((SUB, VD), jnp.float32),
        pltpu.VMEM((SUB * CTX, VD), jnp.float32),
        pltpu.VMEM((SUB * CTX, VD), jnp.float32),
        pltpu.VMEM((SUB, VD), jnp.float32),
        pltpu.VMEM((GC, VD), jnp.float32),
        pltpu.VMEM((GC, VD), jnp.float32),
        pltpu.VMEM((GC, VD), jnp.float32),
        pltpu.VMEM((GC, VD), jnp.float32),
        pltpu.SemaphoreType.DMA,
        pltpu.SemaphoreType.DMA,
        pltpu.SemaphoreType.DMA,
        pltpu.SemaphoreType.DMA,
        pltpu.SemaphoreType.DMA,
        pltpu.SemaphoreType.DMA,
        pltpu.SemaphoreType.DMA,
        pltpu.SemaphoreType.DMA,
        pltpu.SemaphoreType.DMA,
        pltpu.SemaphoreType.DMA,
        pltpu.SemaphoreType.DMA,
        pltpu.SemaphoreType.DMA,
    ],
)
def _sc_fused(doc3_hbm, ctx3_hbm, tid3_hbm, d_hbm, w_hbm, ot_hbm,
              x_hbm, og_hbm,
              didx_v, cidx_v, tidx_v, drows_a, drows_b, wrows_a, wrows_b,
              acc_v, og0, og1, og2, og3,
              dsem_a, dsem_b, wsem_a, wsem_b,
              osem0, osem1, osem2, osem3,
              owsem0, owsem1, owsem2, owsem3):
    wid = lax.axis_index("s") * 2 + lax.axis_index("c")
    base = wid * NB
    ogbuf = (og0, og1, og2, og3)
    osem = (osem0, osem1, osem2, osem3)
    owsem = (owsem0, owsem1, owsem2, owsem3)
    wrows = (wrows_a, wrows_b)
    wsem = (wsem_a, wsem_b)
    drows = (drows_a, drows_b)
    dsem = (dsem_a, dsem_b)

    def d_fire(s, p):
        pltpu.async_copy(
            d_hbm.at[didx_v.at[0, pl.ds(s * SUB, SUB)]], drows[p], dsem[p])

    def d_drain(s, p):
        pltpu.make_async_copy(
            d_hbm.at[didx_v.at[0, pl.ds(s * SUB, SUB)]], drows[p],
            dsem[p]).wait()

    def w_fire(s, p):
        for j in range(5):
            pltpu.async_copy(
                w_hbm.at[cidx_v.at[s * 5 + j]],
                wrows[p].at[pl.ds(j * IROW, IROW)],
                wsem[p],
            )

    def w_drain(s, p):
        for j in range(5):
            pltpu.make_async_copy(
                w_hbm.at[cidx_v.at[s * 5 + j]],
                wrows[p].at[pl.ds(j * IROW, IROW)],
                wsem[p],
            ).wait()

    def og_fire(c, b):
        pltpu.async_copy(ot_hbm.at[tidx_v.at[c]], ogbuf[b], osem[b])

    def og_gather_drain(c, b):
        pltpu.make_async_copy(ot_hbm.at[tidx_v.at[c]], ogbuf[b], osem[b]).wait()

    def og_write(c, b):
        pltpu.async_copy(ogbuf[b], og_hbm.at[pl.ds(c * B + base, GC)],
                         owsem[b])

    def og_write_drain(c, b):
        pltpu.make_async_copy(ogbuf[b], og_hbm.at[pl.ds(c * B + base, GC)],
                              owsem[b]).wait()

    pltpu.sync_copy(doc3_hbm.at[wid], didx_v)
    d_fire(0, 0)
    d_fire(1, 1)
    pltpu.sync_copy(ctx3_hbm.at[wid], cidx_v)
    pltpu.sync_copy(tid3_hbm.at[:, wid], tidx_v)

    w_fire(0, 0)
    w_fire(1, 1)
    for c in range(NOGB):
        og_fire(c, c)

    @pl.loop(0, NSUB // 2)
    def _(g):
        for p in range(2):
            s = 2 * g + p
            for q in range(2):
                c = 2 * s + q
                bq = (2 * p + q) % NOGB

                @pl.when(c < NOISE)
                def _():
                    og_gather_drain(c, bq)
                    og_write(c, bq)

            d_drain(s, p)
            w_drain(s, p)

            @pl.loop(0, SUB)
            def _(b):
                for ch in range(VD // 16):
                    sl = pl.ds(ch * 16, 16)
                    v = drows[p][b, sl]
                    for j in range(CTX):
                        v = v + wrows[p][b * CTX + j, sl]
                    acc_v[b, sl] = v

            pltpu.sync_copy(acc_v, x_hbm.at[pl.ds(base + s * SUB, SUB)])

            for q in range(2):
                cn = 2 * s + 4 + q
                bq = (2 * p + q) % NOGB

                @pl.when(cn < NOISE)
                def _():
                    og_write_drain(cn - NOGB, bq)
                    og_fire(cn, bq)

            @pl.when(s + 2 < NSUB)
            def _():
                d_fire(s + 2, p)
                w_fire(s + 2, p)

    for b in range(NOGB):
        c = NOISE - NOGB + b
        og_write_drain(c, c % NOGB)


def _score_body(x_ref, og_ref, s_ref):
    x = x_ref[...]
    og = og_ref[...]
    s_ref[...] = jnp.sum(og * x[None, :, :], axis=-1)


def kernel(context_ids, doc_ids, target_noise_ids, D, W, O):
    ctx3 = context_ids.astype(jnp.int32).reshape(NW, CROWS, IROW)
    doc3 = doc_ids.astype(jnp.int32).reshape(NW, 1, 128)
    tid3 = target_noise_ids.astype(jnp.int32).T.reshape(NOISE, NW, GC)

    ot = jnp.transpose(O)

    x, og = _sc_fused(doc3, ctx3, tid3, D, W, ot)

    scores_t = pl.pallas_call(
        _score_body,
        grid=(B // 1024,),
        in_specs=[
            pl.BlockSpec((1024, VD), lambda i: (i, 0)),
            pl.BlockSpec((NOISE, 1024, VD), lambda i: (0, i, 0)),
        ],
        out_specs=pl.BlockSpec((NOISE, 1024), lambda i: (0, i)),
        out_shape=jax.ShapeDtypeStruct((NOISE, B), jnp.float32),
    )(x, og.reshape(NOISE, B, VD))

    return jnp.transpose(scores_t)


# speedup vs baseline: 1.0012x; 1.0012x over previous
import functools

import jax
import jax.numpy as jnp
from jax import lax
from jax.experimental import pallas as pl
from jax.experimental.pallas import tpu as pltpu
from jax.experimental.pallas import tpu_sc as plsc

B = 4096
CTX = 20
NOISE = 26
VD = 128
NW = 32
NB = B // NW
SUB = 8
NSUB = NB // SUB
IROW = 32
CROWS = NB * CTX // IROW

_NG = B * NOISE
GC = 128
NOGB = 4

_MESH = plsc.VectorSubcoreMesh(core_axis_name="c", subcore_axis_name="s")


@functools.partial(
    pl.kernel,
    mesh=_MESH,
    out_type=[
        jax.ShapeDtypeStruct((B, VD), jnp.float32),
        jax.ShapeDtypeStruct((_NG, VD), jnp.float32),
    ],
    scratch_types=[
        pltpu.VMEM((1, 128), jnp.int32),
        pltpu.VMEM((CROWS, IROW), jnp.int32),
        pltpu.VMEM((NOISE, GC), jnp.int32),
        pltpu.VMEM((SUB, VD), jnp.float32),
        pltpu.VMEM((SUB, VD), jnp.float32),
        pltpu.VMEM((SUB * CTX, VD), jnp.float32),
        pltpu.VMEM((SUB * CTX, VD), jnp.float32),
        pltpu.VMEM((SUB, VD), jnp.float32),
        pltpu.VMEM((SUB, VD), jnp.float32),
        pltpu.VMEM((GC, VD), jnp.float32),
        pltpu.VMEM((GC, VD), jnp.float32),
        pltpu.VMEM((GC, VD), jnp.float32),
        pltpu.VMEM((GC, VD), jnp.float32),
        pltpu.SemaphoreType.DMA,
        pltpu.SemaphoreType.DMA,
        pltpu.SemaphoreType.DMA,
        pltpu.SemaphoreType.DMA,
        pltpu.SemaphoreType.DMA,
        pltpu.SemaphoreType.DMA,
        pltpu.SemaphoreType.DMA,
        pltpu.SemaphoreType.DMA,
        pltpu.SemaphoreType.DMA,
        pltpu.SemaphoreType.DMA,
        pltpu.SemaphoreType.DMA,
        pltpu.SemaphoreType.DMA,
        pltpu.SemaphoreType.DMA,
        pltpu.SemaphoreType.DMA,
    ],
)
def _sc_fused(doc3_hbm, ctx3_hbm, tid3_hbm, d_hbm, w_hbm, ot_hbm,
              x_hbm, og_hbm,
              didx_v, cidx_v, tidx_v, drows_a, drows_b, wrows_a, wrows_b,
              acc_a, acc_b, og0, og1, og2, og3,
              dsem_a, dsem_b, wsem_a, wsem_b, xsem_a, xsem_b,
              osem0, osem1, osem2, osem3,
              owsem0, owsem1, owsem2, owsem3):
    wid = lax.axis_index("s") * 2 + lax.axis_index("c")
    base = wid * NB
    ogbuf = (og0, og1, og2, og3)
    osem = (osem0, osem1, osem2, osem3)
    owsem = (owsem0, owsem1, owsem2, owsem3)
    wrows = (wrows_a, wrows_b)
    wsem = (wsem_a, wsem_b)
    drows = (drows_a, drows_b)
    dsem = (dsem_a, dsem_b)
    acc = (acc_a, acc_b)
    xsem = (xsem_a, xsem_b)

    def x_write(s, p):
        pltpu.async_copy(acc[p], x_hbm.at[pl.ds(base + s * SUB, SUB)], xsem[p])

    def x_write_drain(s, p):
        pltpu.make_async_copy(
            acc[p], x_hbm.at[pl.ds(base + s * SUB, SUB)], xsem[p]).wait()

    def d_fire(s, p):
        pltpu.async_copy(
            d_hbm.at[didx_v.at[0, pl.ds(s * SUB, SUB)]], drows[p], dsem[p])

    def d_drain(s, p):
        pltpu.make_async_copy(
            d_hbm.at[didx_v.at[0, pl.ds(s * SUB, SUB)]], drows[p],
            dsem[p]).wait()

    def w_fire(s, p):
        for j in range(5):
            pltpu.async_copy(
                w_hbm.at[cidx_v.at[s * 5 + j]],
                wrows[p].at[pl.ds(j * IROW, IROW)],
                wsem[p],
            )

    def w_drain(s, p):
        for j in range(5):
            pltpu.make_async_copy(
                w_hbm.at[cidx_v.at[s * 5 + j]],
                wrows[p].at[pl.ds(j * IROW, IROW)],
                wsem[p],
            ).wait()

    def og_fire(c, b):
        pltpu.async_copy(ot_hbm.at[tidx_v.at[c]], ogbuf[b], osem[b])

    def og_gather_drain(c, b):
        pltpu.make_async_copy(ot_hbm.at[tidx_v.at[c]], ogbuf[b], osem[b]).wait()

    def og_write(c, b):
        pltpu.async_copy(ogbuf[b], og_hbm.at[pl.ds(c * B + base, GC)],
                         owsem[b])

    def og_write_drain(c, b):
        pltpu.make_async_copy(ogbuf[b], og_hbm.at[pl.ds(c * B + base, GC)],
                              owsem[b]).wait()

    pltpu.sync_copy(doc3_hbm.at[wid], didx_v)
    d_fire(0, 0)
    d_fire(1, 1)
    pltpu.sync_copy(ctx3_hbm.at[wid], cidx_v)
    pltpu.sync_copy(tid3_hbm.at[:, wid], tidx_v)

    w_fire(0, 0)
    w_fire(1, 1)
    for c in range(NOGB):
        og_fire(c, c)

    @pl.loop(0, NSUB // 2)
    def _(g):
        for p in range(2):
            s = 2 * g + p
            for q in range(2):
                cp = 2 * s + 2 + q
                bp = (2 * p + 2 + q) % NOGB

                @pl.when(jnp.logical_and(cp >= NOGB, cp < NOISE))
                def _():
                    og_write_drain(cp - NOGB, bp)
                    og_fire(cp, bp)

            for q in range(2):
                c = 2 * s + q
                bq = (2 * p + q) % NOGB

                @pl.when(c < NOISE)
                def _():
                    og_gather_drain(c, bq)
                    og_write(c, bq)

            d_drain(s, p)
            w_drain(s, p)

            @pl.when(s >= 2)
            def _():
                x_write_drain(s - 2, p)

            @pl.loop(0, SUB)
            def _(b):
                for ch in range(VD // 16):
                    sl = pl.ds(ch * 16, 16)
                    v = drows[p][b, sl]
                    for j in range(CTX):
                        v = v + wrows[p][b * CTX + j, sl]
                    acc[p][b, sl] = v

            x_write(s, p)

            @pl.when(s + 2 < NSUB)
            def _():
                d_fire(s + 2, p)
                w_fire(s + 2, p)

    for b in range(NOGB):
        c = NOISE - NOGB + b
        og_write_drain(c, c % NOGB)
    x_write_drain(NSUB - 2, 0)
    x_write_drain(NSUB - 1, 1)


def _score_body(x_ref, og_ref, s_ref):
    x = x_ref[...]
    og = og_ref[...]
    s_ref[...] = jnp.sum(og * x[None, :, :], axis=-1)


def kernel(context_ids, doc_ids, target_noise_ids, D, W, O):
    ctx3 = context_ids.astype(jnp.int32).reshape(NW, CROWS, IROW)
    doc3 = doc_ids.astype(jnp.int32).reshape(NW, 1, 128)
    tid3 = target_noise_ids.astype(jnp.int32).T.reshape(NOISE, NW, GC)

    ot = jnp.transpose(O)

    x, og = _sc_fused(doc3, ctx3, tid3, D, W, ot)

    scores_t = pl.pallas_call(
        _score_body,
        grid=(B // 1024,),
        in_specs=[
            pl.BlockSpec((1024, VD), lambda i: (i, 0)),
            pl.BlockSpec((NOISE, 1024, VD), lambda i: (0, i, 0)),
        ],
        out_specs=pl.BlockSpec((NOISE, 1024), lambda i: (0, i)),
        out_shape=jax.ShapeDtypeStruct((NOISE, B), jnp.float32),
    )(x, og.reshape(NOISE, B, VD))

    return jnp.transpose(scores_t)


# speedup vs baseline: 1.0223x; 1.0211x over previous
import functools

import jax
import jax.numpy as jnp
from jax import lax
from jax.experimental import pallas as pl
from jax.experimental.pallas import tpu as pltpu
from jax.experimental.pallas import tpu_sc as plsc

B = 4096
CTX = 20
NOISE = 26
VD = 128
NW = 32
NB = B // NW
SUB = 8
NSUB = NB // SUB
IROW = 80
CROWS = NB * CTX // IROW
WSTR = SUB * CTX // IROW

_NG = B * NOISE
GC = 128
NOGB = 4

_MESH = plsc.VectorSubcoreMesh(core_axis_name="c", subcore_axis_name="s")


@functools.partial(
    pl.kernel,
    mesh=_MESH,
    out_type=[
        jax.ShapeDtypeStruct((B, VD), jnp.float32),
        jax.ShapeDtypeStruct((_NG, VD), jnp.float32),
    ],
    scratch_types=[
        pltpu.VMEM((1, 128), jnp.int32),
        pltpu.VMEM((CROWS, IROW), jnp.int32),
        pltpu.VMEM((NOISE, GC), jnp.int32),
        pltpu.VMEM((SUB, VD), jnp.float32),
        pltpu.VMEM((SUB, VD), jnp.float32),
        pltpu.VMEM((SUB * CTX, VD), jnp.float32),
        pltpu.VMEM((SUB * CTX, VD), jnp.float32),
        pltpu.VMEM((SUB, VD), jnp.float32),
        pltpu.VMEM((SUB, VD), jnp.float32),
        pltpu.VMEM((GC, VD), jnp.float32),
        pltpu.VMEM((GC, VD), jnp.float32),
        pltpu.VMEM((GC, VD), jnp.float32),
        pltpu.VMEM((GC, VD), jnp.float32),
        pltpu.SemaphoreType.DMA,
        pltpu.SemaphoreType.DMA,
        pltpu.SemaphoreType.DMA,
        pltpu.SemaphoreType.DMA,
        pltpu.SemaphoreType.DMA,
        pltpu.SemaphoreType.DMA,
        pltpu.SemaphoreType.DMA,
        pltpu.SemaphoreType.DMA,
        pltpu.SemaphoreType.DMA,
        pltpu.SemaphoreType.DMA,
        pltpu.SemaphoreType.DMA,
        pltpu.SemaphoreType.DMA,
        pltpu.SemaphoreType.DMA,
        pltpu.SemaphoreType.DMA,
    ],
)
def _sc_fused(doc3_hbm, ctx3_hbm, tid3_hbm, d_hbm, w_hbm, ot_hbm,
              x_hbm, og_hbm,
              didx_v, cidx_v, tidx_v, drows_a, drows_b, wrows_a, wrows_b,
              acc_a, acc_b, og0, og1, og2, og3,
              dsem_a, dsem_b, wsem_a, wsem_b, xsem_a, xsem_b,
              osem0, osem1, osem2, osem3,
              owsem0, owsem1, owsem2, owsem3):
    wid = lax.axis_index("s") * 2 + lax.axis_index("c")
    base = wid * NB
    ogbuf = (og0, og1, og2, og3)
    osem = (osem0, osem1, osem2, osem3)
    owsem = (owsem0, owsem1, owsem2, owsem3)
    wrows = (wrows_a, wrows_b)
    wsem = (wsem_a, wsem_b)
    drows = (drows_a, drows_b)
    dsem = (dsem_a, dsem_b)
    acc = (acc_a, acc_b)
    xsem = (xsem_a, xsem_b)

    def x_write(s, p):
        pltpu.async_copy(acc[p], x_hbm.at[pl.ds(base + s * SUB, SUB)], xsem[p])

    def x_write_drain(s, p):
        pltpu.make_async_copy(
            acc[p], x_hbm.at[pl.ds(base + s * SUB, SUB)], xsem[p]).wait()

    def d_fire(s, p):
        pltpu.async_copy(
            d_hbm.at[didx_v.at[0, pl.ds(s * SUB, SUB)]], drows[p], dsem[p])

    def d_drain(s, p):
        pltpu.make_async_copy(
            d_hbm.at[didx_v.at[0, pl.ds(s * SUB, SUB)]], drows[p],
            dsem[p]).wait()

    def w_fire(s, p):
        for j in range(WSTR):
            pltpu.async_copy(
                w_hbm.at[cidx_v.at[s * WSTR + j]],
                wrows[p].at[pl.ds(j * IROW, IROW)],
                wsem[p],
            )

    def w_drain(s, p):
        for j in range(WSTR):
            pltpu.make_async_copy(
                w_hbm.at[cidx_v.at[s * WSTR + j]],
                wrows[p].at[pl.ds(j * IROW, IROW)],
                wsem[p],
            ).wait()

    def og_fire(c, b):
        pltpu.async_copy(ot_hbm.at[tidx_v.at[c]], ogbuf[b], osem[b])

    def og_gather_drain(c, b):
        pltpu.make_async_copy(ot_hbm.at[tidx_v.at[c]], ogbuf[b], osem[b]).wait()

    def og_write(c, b):
        pltpu.async_copy(ogbuf[b], og_hbm.at[pl.ds(c * B + base, GC)],
                         owsem[b])

    def og_write_drain(c, b):
        pltpu.make_async_copy(ogbuf[b], og_hbm.at[pl.ds(c * B + base, GC)],
                              owsem[b]).wait()

    pltpu.sync_copy(doc3_hbm.at[wid], didx_v)
    d_fire(0, 0)
    d_fire(1, 1)
    pltpu.sync_copy(ctx3_hbm.at[wid], cidx_v)
    pltpu.sync_copy(tid3_hbm.at[:, wid], tidx_v)

    w_fire(0, 0)
    w_fire(1, 1)
    for c in range(NOGB):
        og_fire(c, c)

    @pl.loop(0, NSUB // 2)
    def _(g):
        for p in range(2):
            s = 2 * g + p
            for q in range(2):
                cp = 2 * s + 2 + q
                bp = (2 * p + 2 + q) % NOGB

                @pl.when(jnp.logical_and(cp >= NOGB, cp < NOISE))
                def _():
                    og_write_drain(cp - NOGB, bp)
                    og_fire(cp, bp)

            for q in range(2):
                c = 2 * s + q
                bq = (2 * p + q) % NOGB

                @pl.when(c < NOISE)
                def _():
                    og_gather_drain(c, bq)
                    og_write(c, bq)

            d_drain(s, p)
            w_drain(s, p)

            @pl.when(s >= 2)
            def _():
                x_write_drain(s - 2, p)

            @pl.loop(0, SUB)
            def _(b):
                for ch in range(VD // 16):
                    sl = pl.ds(ch * 16, 16)
                    v = drows[p][b, sl]
                    for j in range(CTX):
                        v = v + wrows[p][b * CTX + j, sl]
                    acc[p][b, sl] = v

            x_write(s, p)

            @pl.when(s + 2 < NSUB)
            def _():
                d_fire(s + 2, p)
                w_fire(s + 2, p)

    for b in range(NOGB):
        c = NOISE - NOGB + b
        og_write_drain(c, c % NOGB)
    x_write_drain(NSUB - 2, 0)
    x_write_drain(NSUB - 1, 1)


def _score_body(x_ref, og_ref, s_ref):
    x = x_ref[...]
    og = og_ref[...]
    s_ref[...] = jnp.sum(og * x[None, :, :], axis=-1)


def kernel(context_ids, doc_ids, target_noise_ids, D, W, O):
    ctx3 = context_ids.astype(jnp.int32).reshape(NW, CROWS, IROW)
    doc3 = doc_ids.astype(jnp.int32).reshape(NW, 1, 128)
    tid3 = target_noise_ids.astype(jnp.int32).T.reshape(NOISE, NW, GC)

    ot = jnp.transpose(O)

    x, og = _sc_fused(doc3, ctx3, tid3, D, W, ot)

    scores_t = pl.pallas_call(
        _score_body,
        grid=(B // 1024,),
        in_specs=[
            pl.BlockSpec((1024, VD), lambda i: (i, 0)),
            pl.BlockSpec((NOISE, 1024, VD), lambda i: (0, i, 0)),
        ],
        out_specs=pl.BlockSpec((NOISE, 1024), lambda i: (0, i)),
        out_shape=jax.ShapeDtypeStruct((NOISE, B), jnp.float32),
    )(x, og.reshape(NOISE, B, VD))

    return jnp.transpose(scores_t)
